# initial kernel scaffold (unmeasured)
import jax
import jax.numpy as jnp
from jax import lax
from jax.experimental import pallas as pl
from jax.experimental.pallas import tpu as pltpu


def kernel(
    x,
):
    def body(*refs):
        pass

    out_shape = jax.ShapeDtypeStruct(..., jnp.float32)
    return pl.pallas_call(body, out_shape=out_shape)(...)


# baseline (device time: 317776 ns/iter reference)
import jax
import jax.numpy as jnp
from jax import lax
from jax.experimental import pallas as pl
from jax.experimental.pallas import tpu as pltpu

N_Z = 4


def kernel(x):
    m_per, n = x.shape

    def body(x_ref, out_ref, send_sems, recv_sems):
        my_x = lax.axis_index("x")
        my_y = lax.axis_index("y")
        my_z = lax.axis_index("z")
        left = (my_z - 1) % N_Z
        right = (my_z + 1) % N_Z

        barrier_sem = pltpu.get_barrier_semaphore()
        for nbr in (left, right):
            pl.semaphore_signal(
                barrier_sem,
                inc=1,
                device_id=(my_x, my_y, nbr),
                device_id_type=pl.DeviceIdType.MESH,
            )
        pl.semaphore_wait(barrier_sem, 2)

        out_ref[pl.ds(my_z * m_per, m_per), :] = x_ref[:, :].astype(
            out_ref.dtype
        )

        for h in range(N_Z - 1):
            src_chunk = (my_z - h) % N_Z
            rdma = pltpu.make_async_remote_copy(
                src_ref=out_ref.at[pl.ds(src_chunk * m_per, m_per), :],
                dst_ref=out_ref.at[pl.ds(src_chunk * m_per, m_per), :],
                send_sem=send_sems.at[h],
                recv_sem=recv_sems.at[h],
                device_id=(my_x, my_y, right),
                device_id_type=pl.DeviceIdType.MESH,
            )
            rdma.start()
            rdma.wait()

    return pl.pallas_call(
        body,
        out_shape=jax.ShapeDtypeStruct((N_Z * m_per, n), jnp.bfloat16),
        in_specs=[pl.BlockSpec(memory_space=pltpu.VMEM)],
        out_specs=pl.BlockSpec(memory_space=pltpu.VMEM),
        scratch_shapes=[
            pltpu.SemaphoreType.DMA((N_Z - 1,)),
            pltpu.SemaphoreType.DMA((N_Z - 1,)),
        ],
        compiler_params=pltpu.CompilerParams(collective_id=0),
    )(x)


# device time: 193148 ns/iter; 1.6452x vs baseline; 1.6452x over previous
import jax
import jax.numpy as jnp
from jax import lax
from jax.experimental import pallas as pl
from jax.experimental.pallas import tpu as pltpu

N_Z = 4
P = 4


def kernel(x):
    m_per, n = x.shape
    H = m_per // 2
    R = H // P
    NP = (N_Z - 1) * P

    def body(x_ref, out_ref, s_zr, r_zl, s_zl, r_zr, s_xl, r_xl, s_xr, r_xr):
        my_x = lax.axis_index("x")
        my_y = lax.axis_index("y")
        my_z = lax.axis_index("z")
        half = my_x * H
        peer_half = (1 - my_x) * H

        def rows(c, off, i):
            return pl.ds(c * m_per + off + (i % P) * R, R)

        def c_sr(i):
            return my_z - i // P

        def c_rl(i):
            return my_z - 1 - i // P

        def c_sl(i):
            return my_z + i // P

        def c_rr(i):
            return my_z + 1 + i // P

        def S_R(i):
            return (my_z <= N_Z - 2) & (my_z >= i // P)

        def R_L(i):
            return my_z >= 1 + i // P

        def S_L(i):
            return (my_z >= 1) & (my_z + i // P <= N_Z - 1)

        def R_R(i):
            return my_z + 1 + i // P <= N_Z - 1

        def mk(src_rows, dst_rows, ssem, rsem, i, dev):
            return pltpu.make_async_remote_copy(
                src_ref=out_ref.at[src_rows, :],
                dst_ref=out_ref.at[dst_rows, :],
                send_sem=ssem.at[i],
                recv_sem=rsem.at[i],
                device_id=dev,
                device_id_type=pl.DeviceIdType.MESH,
            )

        def send_r(i):
            r = rows(c_sr(i), half, i)
            return mk(r, r, s_zr, r_zl, i, (my_x, my_y, my_z + 1))

        def send_l(i):
            r = rows(c_sl(i), half, i)
            return mk(r, r, s_zl, r_zr, i, (my_x, my_y, my_z - 1))

        def recv_l(i):
            r = rows(c_rl(i), half, i)
            return mk(r, r, s_zr, r_zl, i, (my_x, my_y, my_z))

        def recv_r(i):
            r = rows(c_rr(i), half, i)
            return mk(r, r, s_zl, r_zr, i, (my_x, my_y, my_z))

        def xfwd_l(i):
            r = rows(c_rl(i), half, i)
            return mk(r, r, s_xl, r_xl, i, (1 - my_x, my_y, my_z))

        def xfwd_r(i):
            r = rows(c_rr(i), half, i)
            return mk(r, r, s_xr, r_xr, i, (1 - my_x, my_y, my_z))

        def xrecv_l(i):
            r = rows(c_rl(i), peer_half, i)
            return mk(r, r, s_xl, r_xl, i, (my_x, my_y, my_z))

        def xrecv_r(i):
            r = rows(c_rr(i), peer_half, i)
            return mk(r, r, s_xr, r_xr, i, (my_x, my_y, my_z))

        barrier = pltpu.get_barrier_semaphore()

        @pl.when(my_z >= 1)
        def _():
            pl.semaphore_signal(
                barrier, inc=1, device_id=(my_x, my_y, my_z - 1),
                device_id_type=pl.DeviceIdType.MESH)

        @pl.when(my_z <= N_Z - 2)
        def _():
            pl.semaphore_signal(
                barrier, inc=1, device_id=(my_x, my_y, my_z + 1),
                device_id_type=pl.DeviceIdType.MESH)

        pl.semaphore_signal(
            barrier, inc=1, device_id=(1 - my_x, my_y, my_z),
            device_id_type=pl.DeviceIdType.MESH)
        n_nbrs = 1 + (my_z >= 1).astype(jnp.int32) + (
            my_z <= N_Z - 2).astype(jnp.int32)
        pl.semaphore_wait(barrier, n_nbrs)

        out_ref[pl.ds(my_z * m_per, m_per), :] = x_ref[:, :].astype(
            out_ref.dtype)

        for i in range(P):
            @pl.when(S_R(i))
            def _(i=i):
                send_r(i).start()

            @pl.when(S_L(i))
            def _(i=i):
                send_l(i).start()

        for i in range(P, NP):
            j = i - P

            @pl.when(R_L(j))
            def _(j=j):
                recv_l(j).wait_recv()
                xfwd_l(j).start()

            @pl.when(S_R(i))
            def _(i=i):
                send_r(i).start()

            @pl.when(R_R(j))
            def _(j=j):
                recv_r(j).wait_recv()
                xfwd_r(j).start()

            @pl.when(S_L(i))
            def _(i=i):
                send_l(i).start()

        for j in range(NP - P, NP):
            @pl.when(R_L(j))
            def _(j=j):
                recv_l(j).wait_recv()
                xfwd_l(j).start()

            @pl.when(R_R(j))
            def _(j=j):
                recv_r(j).wait_recv()
                xfwd_r(j).start()

        for i in range(NP):
            @pl.when(R_L(i))
            def _(i=i):
                xrecv_l(i).wait_recv()

            @pl.when(R_R(i))
            def _(i=i):
                xrecv_r(i).wait_recv()

        for i in range(NP):
            @pl.when(S_R(i))
            def _(i=i):
                send_r(i).wait_send()

            @pl.when(S_L(i))
            def _(i=i):
                send_l(i).wait_send()

            @pl.when(R_L(i))
            def _(i=i):
                xfwd_l(i).wait_send()

            @pl.when(R_R(i))
            def _(i=i):
                xfwd_r(i).wait_send()

    return pl.pallas_call(
        body,
        out_shape=jax.ShapeDtypeStruct((N_Z * m_per, n), jnp.bfloat16),
        in_specs=[pl.BlockSpec(memory_space=pltpu.VMEM)],
        out_specs=pl.BlockSpec(memory_space=pltpu.VMEM),
        scratch_shapes=[
            pltpu.SemaphoreType.DMA((NP,)),
            pltpu.SemaphoreType.DMA((NP,)),
            pltpu.SemaphoreType.DMA((NP,)),
            pltpu.SemaphoreType.DMA((NP,)),
            pltpu.SemaphoreType.DMA((NP,)),
            pltpu.SemaphoreType.DMA((NP,)),
            pltpu.SemaphoreType.DMA((NP,)),
            pltpu.SemaphoreType.DMA((NP,)),
        ],
        compiler_params=pltpu.CompilerParams(collective_id=0),
    )(x)


# device time: 192376 ns/iter; 1.6518x vs baseline; 1.0040x over previous
import jax
import jax.numpy as jnp
from jax import lax
from jax.experimental import pallas as pl
from jax.experimental.pallas import tpu as pltpu

N_Z = 4
P = 4


def kernel(x):
    m_per, n = x.shape
    H = m_per // 2
    R = H // P
    NP = (N_Z - 1) * P

    def body(x_ref, out_ref, s_zr, r_zl, s_zl, r_zr, s_xl, r_xl, s_xr, r_xr):
        my_x = lax.axis_index("x")
        my_y = lax.axis_index("y")
        my_z = lax.axis_index("z")
        half = my_x * H
        peer_half = (1 - my_x) * H

        def rows(c, off, i):
            return pl.ds(c * m_per + off + (i % P) * R, R)

        def c_sr(i):
            return my_z - i // P

        def c_rl(i):
            return my_z - 1 - i // P

        def c_sl(i):
            return my_z + i // P

        def c_rr(i):
            return my_z + 1 + i // P

        def S_R(i):
            return (my_z <= N_Z - 2) & (my_z >= i // P)

        def R_L(i):
            return my_z >= 1 + i // P

        def S_L(i):
            return (my_z >= 1) & (my_z + i // P <= N_Z - 1)

        def R_R(i):
            return my_z + 1 + i // P <= N_Z - 1

        def mk(src_rows, dst_rows, ssem, rsem, i, dev):
            return pltpu.make_async_remote_copy(
                src_ref=out_ref.at[src_rows, :],
                dst_ref=out_ref.at[dst_rows, :],
                send_sem=ssem.at[i],
                recv_sem=rsem.at[i],
                device_id=dev,
                device_id_type=pl.DeviceIdType.MESH,
            )

        def send_r(i):
            r = rows(c_sr(i), half, i)
            return mk(r, r, s_zr, r_zl, i, (my_x, my_y, my_z + 1))

        def send_l(i):
            r = rows(c_sl(i), half, i)
            return mk(r, r, s_zl, r_zr, i, (my_x, my_y, my_z - 1))

        def recv_l(i):
            r = rows(c_rl(i), half, i)
            return mk(r, r, s_zr, r_zl, i, (my_x, my_y, my_z))

        def recv_r(i):
            r = rows(c_rr(i), half, i)
            return mk(r, r, s_zl, r_zr, i, (my_x, my_y, my_z))

        def xfwd_l(i):
            r = rows(c_rl(i), half, i)
            return mk(r, r, s_xl, r_xl, i, (1 - my_x, my_y, my_z))

        def xfwd_r(i):
            r = rows(c_rr(i), half, i)
            return mk(r, r, s_xr, r_xr, i, (1 - my_x, my_y, my_z))

        def xrecv_l(i):
            r = rows(c_rl(i), peer_half, i)
            return mk(r, r, s_xl, r_xl, i, (my_x, my_y, my_z))

        def xrecv_r(i):
            r = rows(c_rr(i), peer_half, i)
            return mk(r, r, s_xr, r_xr, i, (my_x, my_y, my_z))

        barrier = pltpu.get_barrier_semaphore()

        @pl.when(my_z >= 1)
        def _():
            pl.semaphore_signal(
                barrier, inc=1, device_id=(my_x, my_y, my_z - 1),
                device_id_type=pl.DeviceIdType.MESH)

        @pl.when(my_z <= N_Z - 2)
        def _():
            pl.semaphore_signal(
                barrier, inc=1, device_id=(my_x, my_y, my_z + 1),
                device_id_type=pl.DeviceIdType.MESH)

        pl.semaphore_signal(
            barrier, inc=1, device_id=(1 - my_x, my_y, my_z),
            device_id_type=pl.DeviceIdType.MESH)
        n_nbrs = 1 + (my_z >= 1).astype(jnp.int32) + (
            my_z <= N_Z - 2).astype(jnp.int32)
        pl.semaphore_wait(barrier, n_nbrs)

        for i in range(P):
            out_ref[rows(my_z, half, i), :] = x_ref[
                pl.ds(half + (i % P) * R, R), :].astype(out_ref.dtype)

            @pl.when(S_R(i))
            def _(i=i):
                send_r(i).start()

            @pl.when(S_L(i))
            def _(i=i):
                send_l(i).start()

        out_ref[pl.ds(my_z * m_per + peer_half, H), :] = x_ref[
            pl.ds(peer_half, H), :].astype(out_ref.dtype)

        for i in range(P, NP):
            j = i - P

            @pl.when(R_L(j))
            def _(j=j):
                recv_l(j).wait_recv()
                xfwd_l(j).start()

            @pl.when(S_R(i))
            def _(i=i):
                send_r(i).start()

            @pl.when(R_R(j))
            def _(j=j):
                recv_r(j).wait_recv()
                xfwd_r(j).start()

            @pl.when(S_L(i))
            def _(i=i):
                send_l(i).start()

        for j in range(NP - P, NP):
            @pl.when(R_L(j))
            def _(j=j):
                recv_l(j).wait_recv()
                xfwd_l(j).start()

            @pl.when(R_R(j))
            def _(j=j):
                recv_r(j).wait_recv()
                xfwd_r(j).start()

        for i in range(NP):
            @pl.when(R_L(i))
            def _(i=i):
                xrecv_l(i).wait_recv()

            @pl.when(R_R(i))
            def _(i=i):
                xrecv_r(i).wait_recv()

        for i in range(NP):
            @pl.when(S_R(i))
            def _(i=i):
                send_r(i).wait_send()

            @pl.when(S_L(i))
            def _(i=i):
                send_l(i).wait_send()

            @pl.when(R_L(i))
            def _(i=i):
                xfwd_l(i).wait_send()

            @pl.when(R_R(i))
            def _(i=i):
                xfwd_r(i).wait_send()

    return pl.pallas_call(
        body,
        out_shape=jax.ShapeDtypeStruct((N_Z * m_per, n), jnp.bfloat16),
        in_specs=[pl.BlockSpec(memory_space=pltpu.VMEM)],
        out_specs=pl.BlockSpec(memory_space=pltpu.VMEM),
        scratch_shapes=[
            pltpu.SemaphoreType.DMA((NP,)),
            pltpu.SemaphoreType.DMA((NP,)),
            pltpu.SemaphoreType.DMA((NP,)),
            pltpu.SemaphoreType.DMA((NP,)),
            pltpu.SemaphoreType.DMA((NP,)),
            pltpu.SemaphoreType.DMA((NP,)),
            pltpu.SemaphoreType.DMA((NP,)),
            pltpu.SemaphoreType.DMA((NP,)),
        ],
        compiler_params=pltpu.CompilerParams(collective_id=0),
    )(x)


# device time: 154558 ns/iter; 2.0560x vs baseline; 1.2447x over previous
import jax
import jax.numpy as jnp
from jax import lax
from jax.experimental import pallas as pl
from jax.experimental.pallas import tpu as pltpu

N_Z = 4
P = 4
NP = (N_Z - 1) * P


def kernel(x):
    m_per, n = x.shape
    Q = m_per // 4
    Rq = Q // P

    def body(x_ref, out_ref, *sems):
        (s_zr, r_zl, s_zl, r_zr,
         s_xd0, r_xd0, s_xd1, r_xd1,
         s_yd0, r_yd0, s_yd1, r_yd1,
         s_yr0, r_yr0, s_yr1, r_yr1,
         s_xr0, r_xr0, s_xr1, r_xr1) = sems
        s_xd = (s_xd0, s_xd1); r_xd = (r_xd0, r_xd1)
        s_yd = (s_yd0, s_yd1); r_yd = (r_yd0, r_yd1)
        s_yr = (s_yr0, s_yr1); r_yr = (r_yr0, r_yr1)
        s_xr = (s_xr0, s_xr1); r_xr = (r_xr0, r_xr1)

        my_x = lax.axis_index("x")
        my_y = lax.axis_index("y")
        my_z = lax.axis_index("z")
        yi = my_y % 2
        mate_y = my_y + 1 - 2 * yi
        k = 2 * my_x + yi
        kx = 2 * (1 - my_x) + yi
        ky = 2 * my_x + (1 - yi)
        kd = 3 - k

        dev_self = (my_x, my_y, my_z)
        dev_xp = (1 - my_x, my_y, my_z)
        dev_ym = (my_x, mate_y, my_z)

        def prow(c, qk, i):
            return pl.ds(c * m_per + qk * Q + (i % P) * Rq, Rq)

        def c_of(d, i):
            return (my_z - 1 - i // P) if d == 0 else (my_z + 1 + i // P)

        def R_of(d, i):
            if d == 0:
                return my_z >= 1 + i // P
            return my_z + 1 + i // P <= N_Z - 1

        def S_R(i):
            return (my_z <= N_Z - 2) & (my_z >= i // P)

        def S_L(i):
            return (my_z >= 1) & (my_z + i // P <= N_Z - 1)

        def mk(rowslice, ssem, rsem, i, dev):
            return pltpu.make_async_remote_copy(
                src_ref=out_ref.at[rowslice, :],
                dst_ref=out_ref.at[rowslice, :],
                send_sem=ssem.at[i],
                recv_sem=rsem.at[i],
                device_id=dev,
                device_id_type=pl.DeviceIdType.MESH,
            )

        def send_zr(i):
            return mk(prow(my_z - i // P, k, i), s_zr, r_zl, i,
                      (my_x, my_y, my_z + 1))

        def send_zl(i):
            return mk(prow(my_z + i // P, k, i), s_zl, r_zr, i,
                      (my_x, my_y, my_z - 1))

        def recv_z(d, i):
            ssem, rsem = (s_zr, r_zl) if d == 0 else (s_zl, r_zr)
            return mk(prow(c_of(d, i), k, i), ssem, rsem, i, dev_self)

        def xd_send(d, i):
            return mk(prow(c_of(d, i), k, i), s_xd[d], r_xd[d], i, dev_xp)

        def xd_recv(d, i):
            return mk(prow(c_of(d, i), kx, i), s_xd[d], r_xd[d], i, dev_self)

        def yd_send(d, i):
            return mk(prow(c_of(d, i), k, i), s_yd[d], r_yd[d], i, dev_ym)

        def yd_recv(d, i):
            return mk(prow(c_of(d, i), ky, i), s_yd[d], r_yd[d], i, dev_self)

        def yrel_send(d, i):
            return mk(prow(c_of(d, i), kx, i), s_yr[d], r_yr[d], i, dev_ym)

        def yrel_recv(d, i):
            return mk(prow(c_of(d, i), kd, i), s_yr[d], r_yr[d], i, dev_self)

        def xrel_send(d, i):
            return mk(prow(c_of(d, i), ky, i), s_xr[d], r_xr[d], i, dev_xp)

        def xrel_recv(d, i):
            return mk(prow(c_of(d, i), kd, i), s_xr[d], r_xr[d], i, dev_self)

        barrier = pltpu.get_barrier_semaphore()

        @pl.when(my_z >= 1)
        def _():
            pl.semaphore_signal(
                barrier, inc=1, device_id=(my_x, my_y, my_z - 1),
                device_id_type=pl.DeviceIdType.MESH)

        @pl.when(my_z <= N_Z - 2)
        def _():
            pl.semaphore_signal(
                barrier, inc=1, device_id=(my_x, my_y, my_z + 1),
                device_id_type=pl.DeviceIdType.MESH)

        pl.semaphore_signal(barrier, inc=1, device_id=dev_xp,
                            device_id_type=pl.DeviceIdType.MESH)
        pl.semaphore_signal(barrier, inc=1, device_id=dev_ym,
                            device_id_type=pl.DeviceIdType.MESH)
        n_nbrs = 2 + (my_z >= 1).astype(jnp.int32) + (
            my_z <= N_Z - 2).astype(jnp.int32)
        pl.semaphore_wait(barrier, n_nbrs)

        for i in range(P):
            out_ref[prow(my_z, k, i), :] = x_ref[
                pl.ds(k * Q + (i % P) * Rq, Rq), :].astype(out_ref.dtype)

            @pl.when(S_R(i))
            def _(i=i):
                send_zr(i).start()

            @pl.when(S_L(i))
            def _(i=i):
                send_zl(i).start()

        conv_units = [(o, p) for o in (1, 2, 3) for p in range(P)]
        n_b = NP - P
        conv_per_iter = [len(conv_units) * (t + 1) // n_b for t in range(n_b)]

        def do_conv(u):
            o, pc = conv_units[u]
            qk = (k + o) % 4
            out_ref[prow(my_z, qk, pc), :] = x_ref[
                pl.ds(qk * Q + pc * Rq, Rq), :].astype(out_ref.dtype)

        conv_done = 0
        for i in range(P, NP):
            j = i - P

            @pl.when(R_of(0, j))
            def _(j=j):
                recv_z(0, j).wait_recv()
                xd_send(0, j).start()
                yd_send(0, j).start()

            @pl.when(S_R(i))
            def _(i=i):
                send_zr(i).start()

            @pl.when(R_of(1, j))
            def _(j=j):
                recv_z(1, j).wait_recv()
                xd_send(1, j).start()
                yd_send(1, j).start()

            @pl.when(S_L(i))
            def _(i=i):
                send_zl(i).start()

            j2 = i - 2 * P
            if j2 >= 0:
                for d in (0, 1):
                    @pl.when(R_of(d, j2))
                    def _(d=d, j2=j2):
                        xd_recv(d, j2).wait_recv()
                        if j2 % 2 == 0:
                            yrel_send(d, j2).start()

                    @pl.when(R_of(d, j2))
                    def _(d=d, j2=j2):
                        yd_recv(d, j2).wait_recv()
                        if j2 % 2 == 1:
                            xrel_send(d, j2).start()

            t = i - P
            while conv_done < conv_per_iter[t]:
                do_conv(conv_done)
                conv_done += 1

        for j in range(NP - P, NP):
            for d in (0, 1):
                @pl.when(R_of(d, j))
                def _(d=d, j=j):
                    recv_z(d, j).wait_recv()
                    xd_send(d, j).start()
                    yd_send(d, j).start()

        for j2 in range(max(NP - 2 * P, 0), NP):
            for d in (0, 1):
                @pl.when(R_of(d, j2))
                def _(d=d, j2=j2):
                    xd_recv(d, j2).wait_recv()
                    if j2 % 2 == 0:
                        yrel_send(d, j2).start()

                @pl.when(R_of(d, j2))
                def _(d=d, j2=j2):
                    yd_recv(d, j2).wait_recv()
                    if j2 % 2 == 1:
                        xrel_send(d, j2).start()

        for i in range(NP):
            for d in (0, 1):
                if i % 2 == 0:
                    @pl.when(R_of(d, i))
                    def _(d=d, i=i):
                        yrel_recv(d, i).wait_recv()
                else:
                    @pl.when(R_of(d, i))
                    def _(d=d, i=i):
                        xrel_recv(d, i).wait_recv()

        for i in range(NP):
            @pl.when(S_R(i))
            def _(i=i):
                send_zr(i).wait_send()

            @pl.when(S_L(i))
            def _(i=i):
                send_zl(i).wait_send()

            for d in (0, 1):
                @pl.when(R_of(d, i))
                def _(d=d, i=i):
                    xd_send(d, i).wait_send()
                    yd_send(d, i).wait_send()
                    if i % 2 == 0:
                        yrel_send(d, i).wait_send()
                    else:
                        xrel_send(d, i).wait_send()

    return pl.pallas_call(
        body,
        out_shape=jax.ShapeDtypeStruct((N_Z * m_per, n), jnp.bfloat16),
        in_specs=[pl.BlockSpec(memory_space=pltpu.VMEM)],
        out_specs=pl.BlockSpec(memory_space=pltpu.VMEM),
        scratch_shapes=[pltpu.SemaphoreType.DMA((NP,)) for _ in range(20)],
        compiler_params=pltpu.CompilerParams(collective_id=0),
    )(x)


# device time: 145216 ns/iter; 2.1883x vs baseline; 1.0643x over previous
import jax
import jax.numpy as jnp
from jax import lax
from jax.experimental import pallas as pl
from jax.experimental.pallas import tpu as pltpu

N_Z = 4
P = 4
NP = (N_Z - 1) * P
N_SITES = 28


def kernel(x):
    m_per, n = x.shape
    Q = m_per // 4
    Rq = Q // P

    def body(x_ref, out_ref, comm_ref, *sems):
        (s_zr, r_zl, s_zl, r_zr,
         s_xd0, r_xd0, s_xd1, r_xd1,
         s_yd0, r_yd0, s_yd1, r_yd1,
         s_yr0, r_yr0, s_yr1, r_yr1,
         s_xr0, r_xr0, s_xr1, r_xr1, dsem) = sems
        s_xd = (s_xd0, s_xd1); r_xd = (r_xd0, r_xd1)
        s_yd = (s_yd0, s_yd1); r_yd = (r_yd0, r_yd1)
        s_yr = (s_yr0, s_yr1); r_yr = (r_yr0, r_yr1)
        s_xr = (s_xr0, s_xr1); r_xr = (r_xr0, r_xr1)

        my_x = lax.axis_index("x")
        my_y = lax.axis_index("y")
        my_z = lax.axis_index("z")
        yi = my_y % 2
        mate_y = my_y + 1 - 2 * yi
        k = 2 * my_x + yi
        kx = 2 * (1 - my_x) + yi
        ky = 2 * my_x + (1 - yi)
        kd = 3 - k

        dev_self = (my_x, my_y, my_z)
        dev_xp = (1 - my_x, my_y, my_z)
        dev_ym = (my_x, mate_y, my_z)

        def prow(c, qk, i):
            return pl.ds(c * m_per + qk * Q + (i % P) * Rq, Rq)

        def qrow(c, qk):
            return pl.ds(c * m_per + qk * Q, Q)

        def c_of(d, i):
            return (my_z - 1 - i // P) if d == 0 else (my_z + 1 + i // P)

        def R_of(d, i):
            if d == 0:
                return my_z >= 1 + i // P
            return my_z + 1 + i // P <= N_Z - 1

        def S_R(i):
            return (my_z <= N_Z - 2) & (my_z >= i // P)

        def S_L(i):
            return (my_z >= 1) & (my_z + i // P <= N_Z - 1)

        sites = []

        def _site_cq(kind, a, j):
            if kind == "own":
                return my_z, (k + a) % 4
            qk = {"k": k, "kx": kx, "ky": ky, "kd": kd}[kind]
            return c_of(a, j), qk

        def hbm_copy(idx, kind, a, j):
            c, qk = _site_cq(kind, a, j)
            return pltpu.make_async_copy(
                comm_ref.at[qrow(c, qk), :],
                out_ref.at[qrow(c, qk), :],
                dsem.at[idx],
            )

        def hbm_store(kind, a, j=0):
            hbm_copy(len(sites), kind, a, j).start()
            sites.append((kind, a, j))

        def mk(rowslice, ssem, rsem, i, dev):
            return pltpu.make_async_remote_copy(
                src_ref=comm_ref.at[rowslice, :],
                dst_ref=comm_ref.at[rowslice, :],
                send_sem=ssem.at[i],
                recv_sem=rsem.at[i],
                device_id=dev,
                device_id_type=pl.DeviceIdType.MESH,
            )

        def send_zr(i):
            return mk(prow(my_z - i // P, k, i), s_zr, r_zl, i,
                      (my_x, my_y, my_z + 1))

        def send_zl(i):
            return mk(prow(my_z + i // P, k, i), s_zl, r_zr, i,
                      (my_x, my_y, my_z - 1))

        def recv_z(d, i):
            ssem, rsem = (s_zr, r_zl) if d == 0 else (s_zl, r_zr)
            return mk(prow(c_of(d, i), k, i), ssem, rsem, i, dev_self)

        def xd_send(d, i):
            return mk(prow(c_of(d, i), k, i), s_xd[d], r_xd[d], i, dev_xp)

        def xd_recv(d, i):
            return mk(prow(c_of(d, i), kx, i), s_xd[d], r_xd[d], i, dev_self)

        def yd_send(d, i):
            return mk(prow(c_of(d, i), k, i), s_yd[d], r_yd[d], i, dev_ym)

        def yd_recv(d, i):
            return mk(prow(c_of(d, i), ky, i), s_yd[d], r_yd[d], i, dev_self)

        def yrel_send(d, i):
            return mk(prow(c_of(d, i), kx, i), s_yr[d], r_yr[d], i, dev_ym)

        def yrel_recv(d, i):
            return mk(prow(c_of(d, i), kd, i), s_yr[d], r_yr[d], i, dev_self)

        def xrel_send(d, i):
            return mk(prow(c_of(d, i), ky, i), s_xr[d], r_xr[d], i, dev_xp)

        def xrel_recv(d, i):
            return mk(prow(c_of(d, i), kd, i), s_xr[d], r_xr[d], i, dev_self)

        barrier = pltpu.get_barrier_semaphore()

        @pl.when(my_z >= 1)
        def _():
            pl.semaphore_signal(
                barrier, inc=1, device_id=(my_x, my_y, my_z - 1),
                device_id_type=pl.DeviceIdType.MESH)

        @pl.when(my_z <= N_Z - 2)
        def _():
            pl.semaphore_signal(
                barrier, inc=1, device_id=(my_x, my_y, my_z + 1),
                device_id_type=pl.DeviceIdType.MESH)

        pl.semaphore_signal(barrier, inc=1, device_id=dev_xp,
                            device_id_type=pl.DeviceIdType.MESH)
        pl.semaphore_signal(barrier, inc=1, device_id=dev_ym,
                            device_id_type=pl.DeviceIdType.MESH)
        n_nbrs = 2 + (my_z >= 1).astype(jnp.int32) + (
            my_z <= N_Z - 2).astype(jnp.int32)
        pl.semaphore_wait(barrier, n_nbrs)

        for i in range(P):
            comm_ref[prow(my_z, k, i), :] = x_ref[
                pl.ds(k * Q + (i % P) * Rq, Rq), :].astype(comm_ref.dtype)

            @pl.when(S_R(i))
            def _(i=i):
                send_zr(i).start()

            @pl.when(S_L(i))
            def _(i=i):
                send_zl(i).start()

        hbm_store("own", 0)

        conv_units = [(o, p) for o in (1, 2, 3) for p in range(P)]
        n_b = NP - P
        conv_per_iter = [len(conv_units) * (t + 1) // n_b for t in range(n_b)]

        def do_conv(u):
            o, pc = conv_units[u]
            qk = (k + o) % 4
            comm_ref[prow(my_z, qk, pc), :] = x_ref[
                pl.ds(qk * Q + pc * Rq, Rq), :].astype(comm_ref.dtype)
            if pc == P - 1:
                hbm_store("own", o)

        conv_done = 0
        for i in range(P, NP):
            j = i - P

            @pl.when(R_of(0, j))
            def _(j=j):
                recv_z(0, j).wait_recv()
                xd_send(0, j).start()
                yd_send(0, j).start()
                if j % P == P - 1:
                    hbm_store("k", 0, j)

            @pl.when(S_R(i))
            def _(i=i):
                send_zr(i).start()

            @pl.when(R_of(1, j))
            def _(j=j):
                recv_z(1, j).wait_recv()
                xd_send(1, j).start()
                yd_send(1, j).start()
                if j % P == P - 1:
                    hbm_store("k", 1, j)

            @pl.when(S_L(i))
            def _(i=i):
                send_zl(i).start()

            j2 = i - 2 * P
            if j2 >= 0:
                for d in (0, 1):
                    @pl.when(R_of(d, j2))
                    def _(d=d, j2=j2):
                        xd_recv(d, j2).wait_recv()
                        if j2 % 2 == 0:
                            yrel_send(d, j2).start()
                        if j2 % P == P - 1:
                            hbm_store("kx", d, j2)

                    @pl.when(R_of(d, j2))
                    def _(d=d, j2=j2):
                        yd_recv(d, j2).wait_recv()
                        if j2 % 2 == 1:
                            xrel_send(d, j2).start()
                        if j2 % P == P - 1:
                            hbm_store("ky", d, j2)

            t = i - P
            while conv_done < conv_per_iter[t]:
                do_conv(conv_done)
                conv_done += 1

        for j in range(NP - P, NP):
            for d in (0, 1):
                @pl.when(R_of(d, j))
                def _(d=d, j=j):
                    recv_z(d, j).wait_recv()
                    xd_send(d, j).start()
                    yd_send(d, j).start()
                    if j % P == P - 1:
                        hbm_store("k", d, j)

        for j2 in range(max(NP - 2 * P, 0), NP):
            for d in (0, 1):
                @pl.when(R_of(d, j2))
                def _(d=d, j2=j2):
                    xd_recv(d, j2).wait_recv()
                    if j2 % 2 == 0:
                        yrel_send(d, j2).start()
                    if j2 % P == P - 1:
                        hbm_store("kx", d, j2)

                @pl.when(R_of(d, j2))
                def _(d=d, j2=j2):
                    yd_recv(d, j2).wait_recv()
                    if j2 % 2 == 1:
                        xrel_send(d, j2).start()
                    if j2 % P == P - 1:
                        hbm_store("ky", d, j2)

        for i in range(NP):
            for d in (0, 1):
                if i % 2 == 0:
                    @pl.when(R_of(d, i))
                    def _(d=d, i=i):
                        yrel_recv(d, i).wait_recv()
                else:
                    @pl.when(R_of(d, i))
                    def _(d=d, i=i):
                        xrel_recv(d, i).wait_recv()
                        if i % P == P - 1:
                            hbm_store("kd", d, i)

        assert len(sites) == N_SITES, len(sites)
        for idx, (kind, a, j) in enumerate(sites):
            if kind == "own":
                hbm_copy(idx, kind, a, j).wait()
            else:
                @pl.when(R_of(a, j))
                def _(idx=idx, kind=kind, a=a, j=j):
                    hbm_copy(idx, kind, a, j).wait()

        for i in range(NP):
            @pl.when(S_R(i))
            def _(i=i):
                send_zr(i).wait_send()

            @pl.when(S_L(i))
            def _(i=i):
                send_zl(i).wait_send()

            for d in (0, 1):
                @pl.when(R_of(d, i))
                def _(d=d, i=i):
                    xd_send(d, i).wait_send()
                    yd_send(d, i).wait_send()
                    if i % 2 == 0:
                        yrel_send(d, i).wait_send()
                    else:
                        xrel_send(d, i).wait_send()

    return pl.pallas_call(
        body,
        out_shape=jax.ShapeDtypeStruct((N_Z * m_per, n), jnp.bfloat16),
        in_specs=[pl.BlockSpec(memory_space=pltpu.VMEM)],
        out_specs=pl.BlockSpec(memory_space=pl.MemorySpace.ANY),
        scratch_shapes=(
            [pltpu.VMEM((N_Z * m_per, n), jnp.bfloat16)]
            + [pltpu.SemaphoreType.DMA((NP,)) for _ in range(20)]
            + [pltpu.SemaphoreType.DMA((N_SITES,))]
        ),
        compiler_params=pltpu.CompilerParams(collective_id=0),
    )(x)


# device time: 144531 ns/iter; 2.1987x vs baseline; 1.0047x over previous
import jax
import jax.numpy as jnp
from jax import lax
from jax.experimental import pallas as pl
from jax.experimental.pallas import tpu as pltpu

N_Z = 4
P = 8
NP = (N_Z - 1) * P
N_SITES = 28


def kernel(x):
    m_per, n = x.shape
    Q = m_per // 4
    Rq = Q // P

    def body(x_ref, out_ref, comm_ref, *sems):
        (s_zr, r_zl, s_zl, r_zr,
         s_xd0, r_xd0, s_xd1, r_xd1,
         s_yd0, r_yd0, s_yd1, r_yd1,
         s_yr0, r_yr0, s_yr1, r_yr1,
         s_xr0, r_xr0, s_xr1, r_xr1, dsem) = sems
        s_xd = (s_xd0, s_xd1); r_xd = (r_xd0, r_xd1)
        s_yd = (s_yd0, s_yd1); r_yd = (r_yd0, r_yd1)
        s_yr = (s_yr0, s_yr1); r_yr = (r_yr0, r_yr1)
        s_xr = (s_xr0, s_xr1); r_xr = (r_xr0, r_xr1)

        my_x = lax.axis_index("x")
        my_y = lax.axis_index("y")
        my_z = lax.axis_index("z")
        yi = my_y % 2
        mate_y = my_y + 1 - 2 * yi
        k = 2 * my_x + yi
        kx = 2 * (1 - my_x) + yi
        ky = 2 * my_x + (1 - yi)
        kd = 3 - k

        dev_self = (my_x, my_y, my_z)
        dev_xp = (1 - my_x, my_y, my_z)
        dev_ym = (my_x, mate_y, my_z)

        def prow(c, qk, i):
            return pl.ds(c * m_per + qk * Q + (i % P) * Rq, Rq)

        def qrow(c, qk):
            return pl.ds(c * m_per + qk * Q, Q)

        def c_of(d, i):
            return (my_z - 1 - i // P) if d == 0 else (my_z + 1 + i // P)

        def R_of(d, i):
            if d == 0:
                return my_z >= 1 + i // P
            return my_z + 1 + i // P <= N_Z - 1

        def S_R(i):
            return (my_z <= N_Z - 2) & (my_z >= i // P)

        def S_L(i):
            return (my_z >= 1) & (my_z + i // P <= N_Z - 1)

        sites = []

        def _site_cq(kind, a, j):
            if kind == "own":
                return my_z, (k + a) % 4
            qk = {"k": k, "kx": kx, "ky": ky, "kd": kd}[kind]
            return c_of(a, j), qk

        def hbm_copy(idx, kind, a, j):
            c, qk = _site_cq(kind, a, j)
            return pltpu.make_async_copy(
                comm_ref.at[qrow(c, qk), :],
                out_ref.at[qrow(c, qk), :],
                dsem.at[idx],
            )

        def hbm_store(kind, a, j=0):
            hbm_copy(len(sites), kind, a, j).start()
            sites.append((kind, a, j))

        def mk(rowslice, ssem, rsem, i, dev):
            return pltpu.make_async_remote_copy(
                src_ref=comm_ref.at[rowslice, :],
                dst_ref=comm_ref.at[rowslice, :],
                send_sem=ssem.at[i],
                recv_sem=rsem.at[i],
                device_id=dev,
                device_id_type=pl.DeviceIdType.MESH,
            )

        def send_zr(i):
            return mk(prow(my_z - i // P, k, i), s_zr, r_zl, i,
                      (my_x, my_y, my_z + 1))

        def send_zl(i):
            return mk(prow(my_z + i // P, k, i), s_zl, r_zr, i,
                      (my_x, my_y, my_z - 1))

        def recv_z(d, i):
            ssem, rsem = (s_zr, r_zl) if d == 0 else (s_zl, r_zr)
            return mk(prow(c_of(d, i), k, i), ssem, rsem, i, dev_self)

        def xd_send(d, i):
            return mk(prow(c_of(d, i), k, i), s_xd[d], r_xd[d], i, dev_xp)

        def xd_recv(d, i):
            return mk(prow(c_of(d, i), kx, i), s_xd[d], r_xd[d], i, dev_self)

        def yd_send(d, i):
            return mk(prow(c_of(d, i), k, i), s_yd[d], r_yd[d], i, dev_ym)

        def yd_recv(d, i):
            return mk(prow(c_of(d, i), ky, i), s_yd[d], r_yd[d], i, dev_self)

        def yrel_send(d, i):
            return mk(prow(c_of(d, i), kx, i), s_yr[d], r_yr[d], i, dev_ym)

        def yrel_recv(d, i):
            return mk(prow(c_of(d, i), kd, i), s_yr[d], r_yr[d], i, dev_self)

        def xrel_send(d, i):
            return mk(prow(c_of(d, i), ky, i), s_xr[d], r_xr[d], i, dev_xp)

        def xrel_recv(d, i):
            return mk(prow(c_of(d, i), kd, i), s_xr[d], r_xr[d], i, dev_self)

        barrier = pltpu.get_barrier_semaphore()

        @pl.when(my_z >= 1)
        def _():
            pl.semaphore_signal(
                barrier, inc=1, device_id=(my_x, my_y, my_z - 1),
                device_id_type=pl.DeviceIdType.MESH)

        @pl.when(my_z <= N_Z - 2)
        def _():
            pl.semaphore_signal(
                barrier, inc=1, device_id=(my_x, my_y, my_z + 1),
                device_id_type=pl.DeviceIdType.MESH)

        pl.semaphore_signal(barrier, inc=1, device_id=dev_xp,
                            device_id_type=pl.DeviceIdType.MESH)
        pl.semaphore_signal(barrier, inc=1, device_id=dev_ym,
                            device_id_type=pl.DeviceIdType.MESH)
        n_nbrs = 2 + (my_z >= 1).astype(jnp.int32) + (
            my_z <= N_Z - 2).astype(jnp.int32)
        pl.semaphore_wait(barrier, n_nbrs)

        for i in range(P):
            comm_ref[prow(my_z, k, i), :] = x_ref[
                pl.ds(k * Q + (i % P) * Rq, Rq), :].astype(comm_ref.dtype)

            @pl.when(S_R(i))
            def _(i=i):
                send_zr(i).start()

            @pl.when(S_L(i))
            def _(i=i):
                send_zl(i).start()

        hbm_store("own", 0)

        conv_units = [(o, p) for o in (1, 2, 3) for p in range(P)]
        n_b = NP - P
        conv_per_iter = [len(conv_units) * (t + 1) // n_b for t in range(n_b)]

        def do_conv(u):
            o, pc = conv_units[u]
            qk = (k + o) % 4
            comm_ref[prow(my_z, qk, pc), :] = x_ref[
                pl.ds(qk * Q + pc * Rq, Rq), :].astype(comm_ref.dtype)
            if pc == P - 1:
                hbm_store("own", o)

        conv_done = 0
        for i in range(P, NP):
            j = i - P

            @pl.when(R_of(0, j))
            def _(j=j):
                recv_z(0, j).wait_recv()
                xd_send(0, j).start()
                yd_send(0, j).start()
                if j % P == P - 1:
                    hbm_store("k", 0, j)

            @pl.when(S_R(i))
            def _(i=i):
                send_zr(i).start()

            @pl.when(R_of(1, j))
            def _(j=j):
                recv_z(1, j).wait_recv()
                xd_send(1, j).start()
                yd_send(1, j).start()
                if j % P == P - 1:
                    hbm_store("k", 1, j)

            @pl.when(S_L(i))
            def _(i=i):
                send_zl(i).start()

            j2 = i - 2 * P
            if j2 >= 0:
                for d in (0, 1):
                    @pl.when(R_of(d, j2))
                    def _(d=d, j2=j2):
                        xd_recv(d, j2).wait_recv()
                        if j2 % 2 == 0:
                            yrel_send(d, j2).start()
                        if j2 % P == P - 1:
                            hbm_store("kx", d, j2)

                    @pl.when(R_of(d, j2))
                    def _(d=d, j2=j2):
                        yd_recv(d, j2).wait_recv()
                        if j2 % 2 == 1:
                            xrel_send(d, j2).start()
                        if j2 % P == P - 1:
                            hbm_store("ky", d, j2)

            t = i - P
            while conv_done < conv_per_iter[t]:
                do_conv(conv_done)
                conv_done += 1

        for j in range(NP - P, NP):
            for d in (0, 1):
                @pl.when(R_of(d, j))
                def _(d=d, j=j):
                    recv_z(d, j).wait_recv()
                    xd_send(d, j).start()
                    yd_send(d, j).start()
                    if j % P == P - 1:
                        hbm_store("k", d, j)

        for j2 in range(max(NP - 2 * P, 0), NP):
            for d in (0, 1):
                @pl.when(R_of(d, j2))
                def _(d=d, j2=j2):
                    xd_recv(d, j2).wait_recv()
                    if j2 % 2 == 0:
                        yrel_send(d, j2).start()
                    if j2 % P == P - 1:
                        hbm_store("kx", d, j2)

                @pl.when(R_of(d, j2))
                def _(d=d, j2=j2):
                    yd_recv(d, j2).wait_recv()
                    if j2 % 2 == 1:
                        xrel_send(d, j2).start()
                    if j2 % P == P - 1:
                        hbm_store("ky", d, j2)

        for i in range(NP):
            for d in (0, 1):
                if i % 2 == 0:
                    @pl.when(R_of(d, i))
                    def _(d=d, i=i):
                        yrel_recv(d, i).wait_recv()
                else:
                    @pl.when(R_of(d, i))
                    def _(d=d, i=i):
                        xrel_recv(d, i).wait_recv()
                        if i % P == P - 1:
                            hbm_store("kd", d, i)

        assert len(sites) == N_SITES, len(sites)
        for idx, (kind, a, j) in enumerate(sites):
            if kind == "own":
                hbm_copy(idx, kind, a, j).wait()
            else:
                @pl.when(R_of(a, j))
                def _(idx=idx, kind=kind, a=a, j=j):
                    hbm_copy(idx, kind, a, j).wait()

        for i in range(NP):
            @pl.when(S_R(i))
            def _(i=i):
                send_zr(i).wait_send()

            @pl.when(S_L(i))
            def _(i=i):
                send_zl(i).wait_send()

            for d in (0, 1):
                @pl.when(R_of(d, i))
                def _(d=d, i=i):
                    xd_send(d, i).wait_send()
                    yd_send(d, i).wait_send()
                    if i % 2 == 0:
                        yrel_send(d, i).wait_send()
                    else:
                        xrel_send(d, i).wait_send()

    return pl.pallas_call(
        body,
        out_shape=jax.ShapeDtypeStruct((N_Z * m_per, n), jnp.bfloat16),
        in_specs=[pl.BlockSpec(memory_space=pltpu.VMEM)],
        out_specs=pl.BlockSpec(memory_space=pl.MemorySpace.ANY),
        scratch_shapes=(
            [pltpu.VMEM((N_Z * m_per, n), jnp.bfloat16)]
            + [pltpu.SemaphoreType.DMA((NP,)) for _ in range(20)]
            + [pltpu.SemaphoreType.DMA((N_SITES,))]
        ),
        compiler_params=pltpu.CompilerParams(collective_id=0),
    )(x)
